# baseline (speedup 1.0000x reference)
import functools

import jax
import jax.numpy as jnp
from jax import lax
from jax.experimental import pallas as pl
from jax.experimental.pallas import tpu as pltpu
from jax.experimental.pallas import tpu_sc as plsc

N = 10000
E = 320000
D = 128
H = 128
G = 64
C_OUT = 19

NC = 2
NS = 16
NW = NC * NS
K = 128
CH = 80
CHS = CH // 2
EPAD = NW * CH * K
NR = N + 16
ZR = NR // NS
BN = 1000
NBLK = N // BN


def _sc_scatter_body(src_hbm, dst_hbm, hs_hbm, zeros_hbm, out0_hbm, out1_hbm,
                     idx_s, idx_d, rows_a, rows_b, acc, sem_a, sem_b, sem_c):
    c = lax.axis_index("c")
    s = lax.axis_index("s")
    wid = c * NS + s
    row0 = s * ZR
    pltpu.async_copy(dst_hbm.at[wid], idx_d, sem_a)
    pltpu.async_copy(zeros_hbm, acc.at[pl.ds(row0, ZR)], sem_c)
    pltpu.make_async_copy(dst_hbm.at[wid], idx_d, sem_a).wait()
    pltpu.make_async_copy(zeros_hbm, acc.at[pl.ds(row0, ZR)], sem_c).wait()
    plsc.subcore_barrier()

    for g in range(CH // CHS):
        base = g * CHS
        pltpu.sync_copy(src_hbm.at[wid, pl.ds(base, CHS)], idx_s)
        pltpu.async_copy(hs_hbm.at[idx_s.at[0]], rows_a, sem_a)

        def body(j, carry):
            i0 = 2 * j
            pltpu.async_copy(hs_hbm.at[idx_s.at[i0 + 1]], rows_b, sem_b)
            pltpu.make_async_copy(hs_hbm.at[idx_s.at[i0]], rows_a,
                                  sem_a).wait()
            pltpu.sync_copy(rows_a, acc.at[idx_d.at[base + i0]], add=True)

            @pl.when(j < CHS // 2 - 1)
            def _():
                pltpu.async_copy(hs_hbm.at[idx_s.at[i0 + 2]], rows_a, sem_a)

            pltpu.make_async_copy(hs_hbm.at[idx_s.at[i0 + 1]], rows_b,
                                  sem_b).wait()
            pltpu.sync_copy(rows_b, acc.at[idx_d.at[base + i0 + 1]], add=True)
            return carry

        lax.fori_loop(0, CHS // 2, body, 0)

    plsc.subcore_barrier()
    w0 = s * 624

    @pl.when(c == 0)
    def _():
        pltpu.sync_copy(acc.at[pl.ds(w0, 624)], out0_hbm.at[pl.ds(w0, 624)])

        @pl.when(s == NS - 1)
        def _():
            pltpu.sync_copy(acc.at[pl.ds(9984, 16)],
                            out0_hbm.at[pl.ds(9984, 16)])

    @pl.when(c == 1)
    def _():
        pltpu.sync_copy(acc.at[pl.ds(w0, 624)], out1_hbm.at[pl.ds(w0, 624)])

        @pl.when(s == NS - 1)
        def _():
            pltpu.sync_copy(acc.at[pl.ds(9984, 16)],
                            out1_hbm.at[pl.ds(9984, 16)])


_sc_scatter = functools.partial(
    pl.kernel,
    out_type=[jax.ShapeDtypeStruct((N, H), jnp.float32),
              jax.ShapeDtypeStruct((N, H), jnp.float32)],
    mesh=plsc.VectorSubcoreMesh(core_axis_name="c", subcore_axis_name="s"),
    scratch_types=[
        pltpu.VMEM((CHS, K), jnp.int32),
        pltpu.VMEM((CH, K), jnp.int32),
        pltpu.VMEM((K, H), jnp.float32),
        pltpu.VMEM((K, H), jnp.float32),
        pltpu.VMEM_SHARED((NR, H), jnp.float32),
        pltpu.SemaphoreType.DMA,
        pltpu.SemaphoreType.DMA,
        pltpu.SemaphoreType.DMA,
    ],
    compiler_params=pltpu.CompilerParams(use_tc_tiling_on_sc=False),
)(_sc_scatter_body)


def _sc_deg_body(dst_hbm, ones_hbm, zeros_hbm, out0_hbm, out1_hbm,
                 idx_d, onesb, acc, sem):
    c = lax.axis_index("c")
    s = lax.axis_index("s")
    wid = c * NS + s
    pltpu.sync_copy(dst_hbm.at[wid], idx_d)
    pltpu.sync_copy(ones_hbm, onesb)
    row0 = s * ZR
    pltpu.sync_copy(zeros_hbm, acc.at[pl.ds(row0, ZR)])
    plsc.subcore_barrier()

    def body(i, carry):
        pltpu.sync_copy(onesb, acc.at[idx_d.at[i]], add=True)
        return carry

    lax.fori_loop(0, CH, body, 0)
    plsc.subcore_barrier()
    w0 = s * 624

    @pl.when(c == 0)
    def _():
        pltpu.sync_copy(acc.at[pl.ds(w0, 624)], out0_hbm.at[pl.ds(w0, 624)])

        @pl.when(s == NS - 1)
        def _():
            pltpu.sync_copy(acc.at[pl.ds(9984, 16)],
                            out0_hbm.at[pl.ds(9984, 16)])

    @pl.when(c == 1)
    def _():
        pltpu.sync_copy(acc.at[pl.ds(w0, 624)], out1_hbm.at[pl.ds(w0, 624)])

        @pl.when(s == NS - 1)
        def _():
            pltpu.sync_copy(acc.at[pl.ds(9984, 16)],
                            out1_hbm.at[pl.ds(9984, 16)])


_sc_deg = functools.partial(
    pl.kernel,
    out_type=[jax.ShapeDtypeStruct((N, 16), jnp.float32),
              jax.ShapeDtypeStruct((N, 16), jnp.float32)],
    mesh=plsc.VectorSubcoreMesh(core_axis_name="c", subcore_axis_name="s"),
    scratch_types=[
        pltpu.VMEM((CH, K), jnp.int32),
        pltpu.VMEM((K, 16), jnp.float32),
        pltpu.VMEM_SHARED((NR, 16), jnp.float32),
        pltpu.SemaphoreType.DMA,
    ],
    compiler_params=pltpu.CompilerParams(use_tc_tiling_on_sc=False),
)(_sc_deg_body)


def _tc_b_body(deg0_ref, deg1_ref, x_ref, w1_ref, dis_ref, hs_ref):
    deg = deg0_ref[...] + deg1_ref[...] + 1.0
    dis = lax.rsqrt(deg)
    dis_ref[...] = dis
    xw = jnp.dot(x_ref[...], w1_ref[...], preferred_element_type=jnp.float32)
    hs_ref[...] = xw * dis[:, 0:1]


_tc_b = pl.pallas_call(
    _tc_b_body,
    grid=(NBLK,),
    in_specs=[
        pl.BlockSpec((BN, 16), lambda i: (i, 0)),
        pl.BlockSpec((BN, 16), lambda i: (i, 0)),
        pl.BlockSpec((BN, D), lambda i: (i, 0)),
        pl.BlockSpec((D, H), lambda i: (0, 0)),
    ],
    out_specs=[
        pl.BlockSpec((BN, 16), lambda i: (i, 0)),
        pl.BlockSpec((BN, H), lambda i: (i, 0)),
    ],
    out_shape=[
        jax.ShapeDtypeStruct((N, 16), jnp.float32),
        jax.ShapeDtypeStruct((N, H), jnp.float32),
    ],
)


def _tc_d_body(p0_ref, p1_ref, hs1_ref, dis_ref, b1_ref, w2_ref, hs2_ref):
    dis = dis_ref[...][:, 0:1]
    acc = p0_ref[...] + p1_ref[...] + hs1_ref[...]
    h1 = jnp.maximum(acc * dis + b1_ref[...], 0.0)
    hw = jnp.dot(h1, w2_ref[...], preferred_element_type=jnp.float32)
    hs2_ref[...] = hw * dis


_tc_d = pl.pallas_call(
    _tc_d_body,
    grid=(NBLK,),
    in_specs=[
        pl.BlockSpec((BN, H), lambda i: (i, 0)),
        pl.BlockSpec((BN, H), lambda i: (i, 0)),
        pl.BlockSpec((BN, H), lambda i: (i, 0)),
        pl.BlockSpec((BN, 16), lambda i: (i, 0)),
        pl.BlockSpec((1, H), lambda i: (0, 0)),
        pl.BlockSpec((H, H), lambda i: (0, 0)),
    ],
    out_specs=pl.BlockSpec((BN, H), lambda i: (i, 0)),
    out_shape=jax.ShapeDtypeStruct((N, H), jnp.float32),
)


def _tc_f_body(q0_ref, q1_ref, hs2_ref, dis_ref, b2_ref, batch_ref,
               wo_ref, bo_ref, out_ref, sum_ref, cnt_ref):
    g = pl.program_id(0)
    dis = dis_ref[...][:, 0:1]
    h2 = (q0_ref[...] + q1_ref[...] + hs2_ref[...]) * dis + b2_ref[...]
    bblk = batch_ref[0]
    gids = lax.broadcasted_iota(jnp.int32, (G, BN), 0)
    oh = (gids == bblk).astype(jnp.float32)
    psum = jnp.dot(oh, h2, preferred_element_type=jnp.float32)
    pcnt = jnp.broadcast_to(jnp.sum(oh, axis=1, keepdims=True), (G, H))

    @pl.when(g == 0)
    def _():
        sum_ref[...] = jnp.zeros_like(sum_ref)
        cnt_ref[...] = jnp.zeros_like(cnt_ref)

    sum_ref[...] += psum
    cnt_ref[...] += pcnt

    @pl.when(g == NBLK - 1)
    def _():
        pooled = sum_ref[...] / jnp.maximum(cnt_ref[...], 1.0)
        out_ref[...] = (
            jnp.dot(pooled, wo_ref[...], preferred_element_type=jnp.float32)
            + bo_ref[...]
        )


_tc_f = pl.pallas_call(
    _tc_f_body,
    grid=(NBLK,),
    in_specs=[
        pl.BlockSpec((BN, H), lambda i: (i, 0)),
        pl.BlockSpec((BN, H), lambda i: (i, 0)),
        pl.BlockSpec((BN, H), lambda i: (i, 0)),
        pl.BlockSpec((BN, 16), lambda i: (i, 0)),
        pl.BlockSpec((1, H), lambda i: (0, 0)),
        pl.BlockSpec((1, 1, BN), lambda i: (i, 0, 0)),
        pl.BlockSpec((H, 128), lambda i: (0, 0)),
        pl.BlockSpec((1, 128), lambda i: (0, 0)),
    ],
    out_specs=pl.BlockSpec((G, 128), lambda i: (0, 0)),
    out_shape=jax.ShapeDtypeStruct((G, 128), jnp.float32),
    scratch_shapes=[
        pltpu.VMEM((G, H), jnp.float32),
        pltpu.VMEM((G, H), jnp.float32),
    ],
)


def kernel(x, edge_index, batch, W1, b1, W2, b2, Wo, bo):
    src = jnp.pad(edge_index[0], (0, EPAD - E)).reshape(NW, CH, K)
    dst = jnp.pad(edge_index[1], (0, EPAD - E),
                  constant_values=N).reshape(NW, CH, K)
    zeros_h = jnp.zeros((ZR, H), jnp.float32)
    zeros16 = jnp.zeros((ZR, 16), jnp.float32)
    ones16 = jnp.ones((K, 16), jnp.float32)

    deg0, deg1 = _sc_deg(dst, ones16, zeros16)
    dis16, hs1 = _tc_b(deg0, deg1, x, W1)
    p0, p1 = _sc_scatter(src, dst, hs1, zeros_h)
    hs2 = _tc_d(p0, p1, hs1, dis16, b1.reshape(1, H), W2)
    q0, q1 = _sc_scatter(src, dst, hs2, zeros_h)
    wo_pad = jnp.pad(Wo, ((0, 0), (0, 128 - C_OUT)))
    bo_pad = jnp.pad(bo, (0, 128 - C_OUT)).reshape(1, 128)
    outp = _tc_f(q0, q1, hs2, dis16, b2.reshape(1, H),
                 batch.reshape(NBLK, 1, BN), wo_pad, bo_pad)
    return outp[:, :C_OUT]


# speedup vs baseline: 1.0014x; 1.0014x over previous
import functools

import jax
import jax.numpy as jnp
from jax import lax
from jax.experimental import pallas as pl
from jax.experimental.pallas import tpu as pltpu
from jax.experimental.pallas import tpu_sc as plsc

N = 10000
E = 320000
D = 128
H = 128
G = 64
C_OUT = 19

NC = 2
NS = 16
NW = NC * NS
K = 128
CH = 80
CHS = CH // 2
EPAD = NW * CH * K
NPAD = EPAD - E
NR = N
ZR = NR // NS
BN = 1000
NBLK = N // BN


def _sc_scatter_body(src_hbm, dst_hbm, hs_hbm, zeros_hbm, out0_hbm, out1_hbm,
                     idx_s, idx_d, rows_a, rows_b, acc, sem_a, sem_b, sem_c):
    c = lax.axis_index("c")
    s = lax.axis_index("s")
    wid = c * NS + s
    row0 = s * ZR
    pltpu.async_copy(dst_hbm.at[wid], idx_d, sem_a)
    pltpu.async_copy(zeros_hbm, acc.at[pl.ds(row0, ZR)], sem_c)
    pltpu.make_async_copy(dst_hbm.at[wid], idx_d, sem_a).wait()
    pltpu.make_async_copy(zeros_hbm, acc.at[pl.ds(row0, ZR)], sem_c).wait()
    plsc.subcore_barrier()

    for g in range(CH // CHS):
        base = g * CHS
        pltpu.sync_copy(src_hbm.at[wid, pl.ds(base, CHS)], idx_s)
        pltpu.async_copy(hs_hbm.at[idx_s.at[0]], rows_a, sem_a)

        def body(j, carry):
            i0 = 2 * j
            pltpu.async_copy(hs_hbm.at[idx_s.at[i0 + 1]], rows_b, sem_b)
            pltpu.make_async_copy(hs_hbm.at[idx_s.at[i0]], rows_a,
                                  sem_a).wait()
            pltpu.sync_copy(rows_a, acc.at[idx_d.at[base + i0]], add=True)

            @pl.when(j < CHS // 2 - 1)
            def _():
                pltpu.async_copy(hs_hbm.at[idx_s.at[i0 + 2]], rows_a, sem_a)

            pltpu.make_async_copy(hs_hbm.at[idx_s.at[i0 + 1]], rows_b,
                                  sem_b).wait()
            pltpu.sync_copy(rows_b, acc.at[idx_d.at[base + i0 + 1]], add=True)
            return carry

        lax.fori_loop(0, CHS // 2, body, 0)

    plsc.subcore_barrier()
    w0 = s * 624

    @pl.when(c == 0)
    def _():
        pltpu.sync_copy(acc.at[pl.ds(w0, 624)], out0_hbm.at[pl.ds(w0, 624)])

        @pl.when(s == NS - 1)
        def _():
            pltpu.sync_copy(acc.at[pl.ds(9984, 16)],
                            out0_hbm.at[pl.ds(9984, 16)])

    @pl.when(c == 1)
    def _():
        pltpu.sync_copy(acc.at[pl.ds(w0, 624)], out1_hbm.at[pl.ds(w0, 624)])

        @pl.when(s == NS - 1)
        def _():
            pltpu.sync_copy(acc.at[pl.ds(9984, 16)],
                            out1_hbm.at[pl.ds(9984, 16)])


_sc_scatter = functools.partial(
    pl.kernel,
    out_type=[jax.ShapeDtypeStruct((N, H), jnp.float32),
              jax.ShapeDtypeStruct((N, H), jnp.float32)],
    mesh=plsc.VectorSubcoreMesh(core_axis_name="c", subcore_axis_name="s"),
    scratch_types=[
        pltpu.VMEM((CHS, K), jnp.int32),
        pltpu.VMEM((CH, K), jnp.int32),
        pltpu.VMEM((K, H), jnp.float32),
        pltpu.VMEM((K, H), jnp.float32),
        pltpu.VMEM_SHARED((NR, H), jnp.float32),
        pltpu.SemaphoreType.DMA,
        pltpu.SemaphoreType.DMA,
        pltpu.SemaphoreType.DMA,
    ],
    compiler_params=pltpu.CompilerParams(use_tc_tiling_on_sc=False),
)(_sc_scatter_body)


def _sc_deg_body(dst_hbm, ones_hbm, zeros_hbm, out0_hbm, out1_hbm,
                 idx_d, onesb, acc, sem):
    c = lax.axis_index("c")
    s = lax.axis_index("s")
    wid = c * NS + s
    pltpu.sync_copy(dst_hbm.at[wid], idx_d)
    pltpu.sync_copy(ones_hbm, onesb)
    row0 = s * ZR
    pltpu.sync_copy(zeros_hbm, acc.at[pl.ds(row0, ZR)])
    plsc.subcore_barrier()

    def body(i, carry):
        pltpu.sync_copy(onesb, acc.at[idx_d.at[i]], add=True)
        return carry

    lax.fori_loop(0, CH, body, 0)
    plsc.subcore_barrier()
    w0 = s * 624

    @pl.when(c == 0)
    def _():
        pltpu.sync_copy(acc.at[pl.ds(w0, 624)], out0_hbm.at[pl.ds(w0, 624)])

        @pl.when(s == NS - 1)
        def _():
            pltpu.sync_copy(acc.at[pl.ds(9984, 16)],
                            out0_hbm.at[pl.ds(9984, 16)])

    @pl.when(c == 1)
    def _():
        pltpu.sync_copy(acc.at[pl.ds(w0, 624)], out1_hbm.at[pl.ds(w0, 624)])

        @pl.when(s == NS - 1)
        def _():
            pltpu.sync_copy(acc.at[pl.ds(9984, 16)],
                            out1_hbm.at[pl.ds(9984, 16)])


_sc_deg = functools.partial(
    pl.kernel,
    out_type=[jax.ShapeDtypeStruct((N, 16), jnp.float32),
              jax.ShapeDtypeStruct((N, 16), jnp.float32)],
    mesh=plsc.VectorSubcoreMesh(core_axis_name="c", subcore_axis_name="s"),
    scratch_types=[
        pltpu.VMEM((CH, K), jnp.int32),
        pltpu.VMEM((K, 16), jnp.float32),
        pltpu.VMEM_SHARED((NR, 16), jnp.float32),
        pltpu.SemaphoreType.DMA,
    ],
    compiler_params=pltpu.CompilerParams(use_tc_tiling_on_sc=False),
)(_sc_deg_body)


def _tc_b_body(deg0_ref, deg1_ref, x_ref, w1_ref, dis_ref, hs_ref):
    i = pl.program_id(0)
    row = i * BN + lax.broadcasted_iota(jnp.int32, (BN, 1), 0)
    padc = jnp.where(row < NPAD, 1.0, 0.0)
    deg = deg0_ref[...] + deg1_ref[...] + (1.0 - padc)
    dis = lax.rsqrt(deg)
    dis_ref[...] = dis
    xw = jnp.dot(x_ref[...], w1_ref[...], preferred_element_type=jnp.float32)
    hs_ref[...] = xw * dis[:, 0:1]


_tc_b = pl.pallas_call(
    _tc_b_body,
    grid=(NBLK,),
    in_specs=[
        pl.BlockSpec((BN, 16), lambda i: (i, 0)),
        pl.BlockSpec((BN, 16), lambda i: (i, 0)),
        pl.BlockSpec((BN, D), lambda i: (i, 0)),
        pl.BlockSpec((D, H), lambda i: (0, 0)),
    ],
    out_specs=[
        pl.BlockSpec((BN, 16), lambda i: (i, 0)),
        pl.BlockSpec((BN, H), lambda i: (i, 0)),
    ],
    out_shape=[
        jax.ShapeDtypeStruct((N, 16), jnp.float32),
        jax.ShapeDtypeStruct((N, H), jnp.float32),
    ],
)


def _tc_d_body(p0_ref, p1_ref, hs1_ref, hs1r0_ref, dis_ref, b1_ref, w2_ref,
               hs2_ref):
    i = pl.program_id(0)
    row = i * BN + lax.broadcasted_iota(jnp.int32, (BN, 1), 0)
    padc = jnp.where(row < NPAD, 1.0, 0.0)
    dis = dis_ref[...][:, 0:1]
    acc = (p0_ref[...] + p1_ref[...] + hs1_ref[...]
           - padc * hs1r0_ref[...])
    h1 = jnp.maximum(acc * dis + b1_ref[...], 0.0)
    hw = jnp.dot(h1, w2_ref[...], preferred_element_type=jnp.float32)
    hs2_ref[...] = hw * dis


_tc_d = pl.pallas_call(
    _tc_d_body,
    grid=(NBLK,),
    in_specs=[
        pl.BlockSpec((BN, H), lambda i: (i, 0)),
        pl.BlockSpec((BN, H), lambda i: (i, 0)),
        pl.BlockSpec((BN, H), lambda i: (i, 0)),
        pl.BlockSpec((1, H), lambda i: (0, 0)),
        pl.BlockSpec((BN, 16), lambda i: (i, 0)),
        pl.BlockSpec((1, H), lambda i: (0, 0)),
        pl.BlockSpec((H, H), lambda i: (0, 0)),
    ],
    out_specs=pl.BlockSpec((BN, H), lambda i: (i, 0)),
    out_shape=jax.ShapeDtypeStruct((N, H), jnp.float32),
)


def _tc_f_body(q0_ref, q1_ref, hs2_ref, hs2r0_ref, dis_ref, b2_ref, batch_ref,
               wo_ref, bo_ref, out_ref, sum_ref, cnt_ref):
    g = pl.program_id(0)
    row = g * BN + lax.broadcasted_iota(jnp.int32, (BN, 1), 0)
    padc = jnp.where(row < NPAD, 1.0, 0.0)
    dis = dis_ref[...][:, 0:1]
    h2 = ((q0_ref[...] + q1_ref[...] + hs2_ref[...] - padc * hs2r0_ref[...])
          * dis + b2_ref[...])
    bblk = batch_ref[0]
    gids = lax.broadcasted_iota(jnp.int32, (G, BN), 0)
    oh = (gids == bblk).astype(jnp.float32)
    psum = jnp.dot(oh, h2, preferred_element_type=jnp.float32)
    pcnt = jnp.broadcast_to(jnp.sum(oh, axis=1, keepdims=True), (G, H))

    @pl.when(g == 0)
    def _():
        sum_ref[...] = jnp.zeros_like(sum_ref)
        cnt_ref[...] = jnp.zeros_like(cnt_ref)

    sum_ref[...] += psum
    cnt_ref[...] += pcnt

    @pl.when(g == NBLK - 1)
    def _():
        pooled = sum_ref[...] / jnp.maximum(cnt_ref[...], 1.0)
        out_ref[...] = (
            jnp.dot(pooled, wo_ref[...], preferred_element_type=jnp.float32)
            + bo_ref[...]
        )


_tc_f = pl.pallas_call(
    _tc_f_body,
    grid=(NBLK,),
    in_specs=[
        pl.BlockSpec((BN, H), lambda i: (i, 0)),
        pl.BlockSpec((BN, H), lambda i: (i, 0)),
        pl.BlockSpec((BN, H), lambda i: (i, 0)),
        pl.BlockSpec((1, H), lambda i: (0, 0)),
        pl.BlockSpec((BN, 16), lambda i: (i, 0)),
        pl.BlockSpec((1, H), lambda i: (0, 0)),
        pl.BlockSpec((1, 1, BN), lambda i: (i, 0, 0)),
        pl.BlockSpec((H, 128), lambda i: (0, 0)),
        pl.BlockSpec((1, 128), lambda i: (0, 0)),
    ],
    out_specs=pl.BlockSpec((G, 128), lambda i: (0, 0)),
    out_shape=jax.ShapeDtypeStruct((G, 128), jnp.float32),
    scratch_shapes=[
        pltpu.VMEM((G, H), jnp.float32),
        pltpu.VMEM((G, H), jnp.float32),
    ],
)


def kernel(x, edge_index, batch, W1, b1, W2, b2, Wo, bo):
    src = jnp.pad(edge_index[0], (0, NPAD)).reshape(NW, CH, K)
    dst = jnp.concatenate(
        [edge_index[1], jnp.arange(NPAD, dtype=jnp.int32)]).reshape(NW, CH, K)
    zeros_h = jnp.zeros((ZR, H), jnp.float32)
    zeros16 = jnp.zeros((ZR, 16), jnp.float32)
    ones16 = jnp.ones((K, 16), jnp.float32)

    deg0, deg1 = _sc_deg(dst, ones16, zeros16)
    dis16, hs1 = _tc_b(deg0, deg1, x, W1)
    p0, p1 = _sc_scatter(src, dst, hs1, zeros_h)
    hs2 = _tc_d(p0, p1, hs1, hs1[0:1], dis16, b1.reshape(1, H), W2)
    q0, q1 = _sc_scatter(src, dst, hs2, zeros_h)
    wo_pad = jnp.pad(Wo, ((0, 0), (0, 128 - C_OUT)))
    bo_pad = jnp.pad(bo, (0, 128 - C_OUT)).reshape(1, 128)
    outp = _tc_f(q0, q1, hs2, hs2[0:1], dis16, b2.reshape(1, H),
                 batch.reshape(NBLK, 1, BN), wo_pad, bo_pad)
    return outp[:, :C_OUT]


# speedup vs baseline: 3.2306x; 3.2263x over previous
import functools

import jax
import jax.numpy as jnp
from jax import lax
from jax.experimental import pallas as pl
from jax.experimental.pallas import tpu as pltpu
from jax.experimental.pallas import tpu_sc as plsc

N = 10000
E = 320000
D = 128
H = 128
G = 64
C_OUT = 19

NC = 2
NS = 16
NW = NC * NS
K = 128
CH = 80
CHS = CH // 2
EPAD = NW * CH * K
NPAD = EPAD - E
NR = N
ZR = NR // NS
BN = 1000
NBLK = N // BN


def _sc_scatter_body(src_hbm, dst_hbm, hs_hbm, zeros_hbm, out0_hbm, out1_hbm,
                     idx_s, idx_d, rows_a, rows_b, acc, sem_a, sem_b, sem_c):
    c = lax.axis_index("c")
    s = lax.axis_index("s")
    wid = c * NS + s
    row0 = s * ZR
    pltpu.async_copy(dst_hbm.at[wid], idx_d, sem_a)
    pltpu.async_copy(zeros_hbm, acc.at[pl.ds(row0, ZR)], sem_c)
    pltpu.make_async_copy(dst_hbm.at[wid], idx_d, sem_a).wait()
    pltpu.make_async_copy(zeros_hbm, acc.at[pl.ds(row0, ZR)], sem_c).wait()
    plsc.subcore_barrier()

    for g in range(CH // CHS):
        base = g * CHS
        pltpu.sync_copy(src_hbm.at[wid, pl.ds(base, CHS)], idx_s)
        pltpu.async_copy(hs_hbm.at[idx_s.at[0]], rows_a, sem_a)

        def body(j, carry):
            i0 = 2 * j
            pltpu.async_copy(hs_hbm.at[idx_s.at[i0 + 1]], rows_b, sem_b)
            pltpu.make_async_copy(hs_hbm.at[idx_s.at[i0]], rows_a,
                                  sem_a).wait()
            pltpu.sync_copy(rows_a, acc.at[idx_d.at[base + i0]], add=True)

            @pl.when(j < CHS // 2 - 1)
            def _():
                pltpu.async_copy(hs_hbm.at[idx_s.at[i0 + 2]], rows_a, sem_a)

            pltpu.make_async_copy(hs_hbm.at[idx_s.at[i0 + 1]], rows_b,
                                  sem_b).wait()
            pltpu.sync_copy(rows_b, acc.at[idx_d.at[base + i0 + 1]], add=True)
            return carry

        lax.fori_loop(0, CHS // 2, body, 0)

    plsc.subcore_barrier()
    w0 = s * 624

    @pl.when(c == 0)
    def _():
        pltpu.sync_copy(acc.at[pl.ds(w0, 624)], out0_hbm.at[pl.ds(w0, 624)])

        @pl.when(s == NS - 1)
        def _():
            pltpu.sync_copy(acc.at[pl.ds(9984, 16)],
                            out0_hbm.at[pl.ds(9984, 16)])

    @pl.when(c == 1)
    def _():
        pltpu.sync_copy(acc.at[pl.ds(w0, 624)], out1_hbm.at[pl.ds(w0, 624)])

        @pl.when(s == NS - 1)
        def _():
            pltpu.sync_copy(acc.at[pl.ds(9984, 16)],
                            out1_hbm.at[pl.ds(9984, 16)])


_sc_scatter = functools.partial(
    pl.kernel,
    out_type=[jax.ShapeDtypeStruct((N, H), jnp.float32),
              jax.ShapeDtypeStruct((N, H), jnp.float32)],
    mesh=plsc.VectorSubcoreMesh(core_axis_name="c", subcore_axis_name="s"),
    scratch_types=[
        pltpu.VMEM((CHS, K), jnp.int32),
        pltpu.VMEM((CH, K), jnp.int32),
        pltpu.VMEM((K, H), jnp.float32),
        pltpu.VMEM((K, H), jnp.float32),
        pltpu.VMEM_SHARED((NR, H), jnp.float32),
        pltpu.SemaphoreType.DMA,
        pltpu.SemaphoreType.DMA,
        pltpu.SemaphoreType.DMA,
    ],
    compiler_params=pltpu.CompilerParams(use_tc_tiling_on_sc=False),
)(_sc_scatter_body)


def _sc_deg_body(dst_hbm, ones_hbm, zeros_hbm, out0_hbm, out1_hbm,
                 idx_d, onesb, acc, sem):
    c = lax.axis_index("c")
    s = lax.axis_index("s")
    wid = c * NS + s
    pltpu.sync_copy(dst_hbm.at[wid], idx_d)
    pltpu.sync_copy(ones_hbm, onesb)
    row0 = s * ZR
    pltpu.sync_copy(zeros_hbm, acc.at[pl.ds(row0, ZR)])
    plsc.subcore_barrier()

    def body(i, carry):
        pltpu.sync_copy(onesb, acc.at[idx_d.at[i]], add=True)
        return carry

    lax.fori_loop(0, CH, body, 0)
    plsc.subcore_barrier()
    w0 = s * 624

    @pl.when(c == 0)
    def _():
        pltpu.sync_copy(acc.at[pl.ds(w0, 624)], out0_hbm.at[pl.ds(w0, 624)])

        @pl.when(s == NS - 1)
        def _():
            pltpu.sync_copy(acc.at[pl.ds(9984, 16)],
                            out0_hbm.at[pl.ds(9984, 16)])

    @pl.when(c == 1)
    def _():
        pltpu.sync_copy(acc.at[pl.ds(w0, 624)], out1_hbm.at[pl.ds(w0, 624)])

        @pl.when(s == NS - 1)
        def _():
            pltpu.sync_copy(acc.at[pl.ds(9984, 16)],
                            out1_hbm.at[pl.ds(9984, 16)])


_sc_deg = functools.partial(
    pl.kernel,
    out_type=[jax.ShapeDtypeStruct((N, 16), jnp.float32),
              jax.ShapeDtypeStruct((N, 16), jnp.float32)],
    mesh=plsc.VectorSubcoreMesh(core_axis_name="c", subcore_axis_name="s"),
    scratch_types=[
        pltpu.VMEM((CH, K), jnp.int32),
        pltpu.VMEM((K, 16), jnp.float32),
        pltpu.VMEM_SHARED((NR, 16), jnp.float32),
        pltpu.SemaphoreType.DMA,
    ],
    compiler_params=pltpu.CompilerParams(use_tc_tiling_on_sc=False),
)(_sc_deg_body)


def _tc_b_body(deg0_ref, deg1_ref, x_ref, w1_ref, dis_ref, hs_ref):
    i = pl.program_id(0)
    row = i * BN + lax.broadcasted_iota(jnp.int32, (BN, 1), 0)
    padc = jnp.where(row < NPAD, 1.0, 0.0)
    deg = deg0_ref[...] + deg1_ref[...] + (1.0 - padc)
    dis = lax.rsqrt(deg)
    dis_ref[...] = dis
    xw = jnp.dot(x_ref[...], w1_ref[...], preferred_element_type=jnp.float32)
    hs_ref[...] = xw * dis[:, 0:1]


_tc_b = pl.pallas_call(
    _tc_b_body,
    grid=(NBLK,),
    in_specs=[
        pl.BlockSpec((BN, 16), lambda i: (i, 0)),
        pl.BlockSpec((BN, 16), lambda i: (i, 0)),
        pl.BlockSpec((BN, D), lambda i: (i, 0)),
        pl.BlockSpec((D, H), lambda i: (0, 0)),
    ],
    out_specs=[
        pl.BlockSpec((BN, 16), lambda i: (i, 0)),
        pl.BlockSpec((BN, H), lambda i: (i, 0)),
    ],
    out_shape=[
        jax.ShapeDtypeStruct((N, 16), jnp.float32),
        jax.ShapeDtypeStruct((N, H), jnp.float32),
    ],
)


def _tc_d_body(p0_ref, p1_ref, hs1_ref, dis_ref, b1_ref, w2_ref, hs2_ref):
    i = pl.program_id(0)
    row = i * BN + lax.broadcasted_iota(jnp.int32, (BN, 1), 0)
    self_w = jnp.where(row < NPAD, 0.0, 1.0)
    dis = dis_ref[...][:, 0:1]
    acc = p0_ref[...] + p1_ref[...] + self_w * hs1_ref[...]
    h1 = jnp.maximum(acc * dis + b1_ref[...], 0.0)
    hw = jnp.dot(h1, w2_ref[...], preferred_element_type=jnp.float32)
    hs2_ref[...] = hw * dis


_tc_d = pl.pallas_call(
    _tc_d_body,
    grid=(NBLK,),
    in_specs=[
        pl.BlockSpec((BN, H), lambda i: (i, 0)),
        pl.BlockSpec((BN, H), lambda i: (i, 0)),
        pl.BlockSpec((BN, H), lambda i: (i, 0)),
        pl.BlockSpec((BN, 16), lambda i: (i, 0)),
        pl.BlockSpec((1, H), lambda i: (0, 0)),
        pl.BlockSpec((H, H), lambda i: (0, 0)),
    ],
    out_specs=pl.BlockSpec((BN, H), lambda i: (i, 0)),
    out_shape=jax.ShapeDtypeStruct((N, H), jnp.float32),
)


def _tc_f_body(q0_ref, q1_ref, hs2_ref, dis_ref, b2_ref, batch_ref,
               wo_ref, bo_ref, out_ref, sum_ref, cnt_ref):
    g = pl.program_id(0)
    row = g * BN + lax.broadcasted_iota(jnp.int32, (BN, 1), 0)
    self_w = jnp.where(row < NPAD, 0.0, 1.0)
    dis = dis_ref[...][:, 0:1]
    h2 = ((q0_ref[...] + q1_ref[...] + self_w * hs2_ref[...])
          * dis + b2_ref[...])
    bblk = batch_ref[0]
    gids = lax.broadcasted_iota(jnp.int32, (G, BN), 0)
    oh = (gids == bblk).astype(jnp.float32)
    psum = jnp.dot(oh, h2, preferred_element_type=jnp.float32)
    pcnt = jnp.broadcast_to(jnp.sum(oh, axis=1, keepdims=True), (G, H))

    @pl.when(g == 0)
    def _():
        sum_ref[...] = jnp.zeros_like(sum_ref)
        cnt_ref[...] = jnp.zeros_like(cnt_ref)

    sum_ref[...] += psum
    cnt_ref[...] += pcnt

    @pl.when(g == NBLK - 1)
    def _():
        pooled = sum_ref[...] / jnp.maximum(cnt_ref[...], 1.0)
        out_ref[...] = (
            jnp.dot(pooled, wo_ref[...], preferred_element_type=jnp.float32)
            + bo_ref[...]
        )


_tc_f = pl.pallas_call(
    _tc_f_body,
    grid=(NBLK,),
    in_specs=[
        pl.BlockSpec((BN, H), lambda i: (i, 0)),
        pl.BlockSpec((BN, H), lambda i: (i, 0)),
        pl.BlockSpec((BN, H), lambda i: (i, 0)),
        pl.BlockSpec((BN, 16), lambda i: (i, 0)),
        pl.BlockSpec((1, H), lambda i: (0, 0)),
        pl.BlockSpec((1, 1, BN), lambda i: (i, 0, 0)),
        pl.BlockSpec((H, 128), lambda i: (0, 0)),
        pl.BlockSpec((1, 128), lambda i: (0, 0)),
    ],
    out_specs=pl.BlockSpec((G, 128), lambda i: (0, 0)),
    out_shape=jax.ShapeDtypeStruct((G, 128), jnp.float32),
    scratch_shapes=[
        pltpu.VMEM((G, H), jnp.float32),
        pltpu.VMEM((G, H), jnp.float32),
    ],
)


def kernel(x, edge_index, batch, W1, b1, W2, b2, Wo, bo):
    pad_idx = jnp.arange(NPAD, dtype=jnp.int32)
    src = jnp.concatenate([edge_index[0], pad_idx]).reshape(NW, CH, K)
    dst = jnp.concatenate([edge_index[1], pad_idx]).reshape(NW, CH, K)
    zeros_h = jnp.zeros((ZR, H), jnp.float32)
    zeros16 = jnp.zeros((ZR, 16), jnp.float32)
    ones16 = jnp.ones((K, 16), jnp.float32)

    deg0, deg1 = _sc_deg(dst, ones16, zeros16)
    dis16, hs1 = _tc_b(deg0, deg1, x, W1)
    p0, p1 = _sc_scatter(src, dst, hs1, zeros_h)
    hs2 = _tc_d(p0, p1, hs1, dis16, b1.reshape(1, H), W2)
    q0, q1 = _sc_scatter(src, dst, hs2, zeros_h)
    wo_pad = jnp.pad(Wo, ((0, 0), (0, 128 - C_OUT)))
    bo_pad = jnp.pad(bo, (0, 128 - C_OUT)).reshape(1, 128)
    outp = _tc_f(q0, q1, hs2, dis16, b2.reshape(1, H),
                 batch.reshape(NBLK, 1, BN), wo_pad, bo_pad)
    return outp[:, :C_OUT]


# speedup vs baseline: 3.2982x; 1.0209x over previous
import functools

import jax
import jax.numpy as jnp
from jax import lax
from jax.experimental import pallas as pl
from jax.experimental.pallas import tpu as pltpu
from jax.experimental.pallas import tpu_sc as plsc

N = 10000
E = 320000
D = 128
H = 128
G = 64
C_OUT = 19

NC = 2
NS = 16
NW = NC * NS
K = 128
CH = 80
CHS = CH // 2
EPAD = NW * CH * K
NPAD = EPAD - E
NR = N
ZR = NR // NS
BN = 1000
NBLK = N // BN


def _sc_scatter_body(edges_hbm, hs_hbm, zeros_hbm, out0_hbm, out1_hbm,
                     idx_s, idx_d, rows_a, rows_b, acc, sem_a, sem_b, sem_c):
    c = lax.axis_index("c")
    s = lax.axis_index("s")
    wid = c * NS + s
    row0 = s * ZR
    pltpu.async_copy(edges_hbm.at[1, wid], idx_d, sem_a)
    pltpu.async_copy(zeros_hbm, acc.at[pl.ds(row0, ZR)], sem_c)
    pltpu.make_async_copy(edges_hbm.at[1, wid], idx_d, sem_a).wait()
    pltpu.make_async_copy(zeros_hbm, acc.at[pl.ds(row0, ZR)], sem_c).wait()
    plsc.subcore_barrier()

    for g in range(CH // CHS):
        base = g * CHS
        pltpu.sync_copy(edges_hbm.at[0, wid, pl.ds(base, CHS)], idx_s)
        pltpu.async_copy(hs_hbm.at[idx_s.at[0]], rows_a, sem_a)

        def body(j, carry):
            i0 = 2 * j
            pltpu.async_copy(hs_hbm.at[idx_s.at[i0 + 1]], rows_b, sem_b)
            pltpu.make_async_copy(hs_hbm.at[idx_s.at[i0]], rows_a,
                                  sem_a).wait()
            pltpu.sync_copy(rows_a, acc.at[idx_d.at[base + i0]], add=True)

            @pl.when(j < CHS // 2 - 1)
            def _():
                pltpu.async_copy(hs_hbm.at[idx_s.at[i0 + 2]], rows_a, sem_a)

            pltpu.make_async_copy(hs_hbm.at[idx_s.at[i0 + 1]], rows_b,
                                  sem_b).wait()
            pltpu.sync_copy(rows_b, acc.at[idx_d.at[base + i0 + 1]], add=True)
            return carry

        lax.fori_loop(0, CHS // 2, body, 0)

    plsc.subcore_barrier()
    w0 = s * 624

    @pl.when(c == 0)
    def _():
        pltpu.sync_copy(acc.at[pl.ds(w0, 624)], out0_hbm.at[pl.ds(w0, 624)])

        @pl.when(s == NS - 1)
        def _():
            pltpu.sync_copy(acc.at[pl.ds(9984, 16)],
                            out0_hbm.at[pl.ds(9984, 16)])

    @pl.when(c == 1)
    def _():
        pltpu.sync_copy(acc.at[pl.ds(w0, 624)], out1_hbm.at[pl.ds(w0, 624)])

        @pl.when(s == NS - 1)
        def _():
            pltpu.sync_copy(acc.at[pl.ds(9984, 16)],
                            out1_hbm.at[pl.ds(9984, 16)])


_sc_scatter = functools.partial(
    pl.kernel,
    out_type=[jax.ShapeDtypeStruct((N, H), jnp.float32),
              jax.ShapeDtypeStruct((N, H), jnp.float32)],
    mesh=plsc.VectorSubcoreMesh(core_axis_name="c", subcore_axis_name="s"),
    scratch_types=[
        pltpu.VMEM((CHS, K), jnp.int32),
        pltpu.VMEM((CH, K), jnp.int32),
        pltpu.VMEM((K, H), jnp.float32),
        pltpu.VMEM((K, H), jnp.float32),
        pltpu.VMEM_SHARED((NR, H), jnp.float32),
        pltpu.SemaphoreType.DMA,
        pltpu.SemaphoreType.DMA,
        pltpu.SemaphoreType.DMA,
    ],
    compiler_params=pltpu.CompilerParams(use_tc_tiling_on_sc=False),
)(_sc_scatter_body)


def _sc_deg_body(edges_hbm, ones_hbm, zeros_hbm, out0_hbm, out1_hbm,
                 idx_d, onesb, acc, sem):
    c = lax.axis_index("c")
    s = lax.axis_index("s")
    wid = c * NS + s
    pltpu.sync_copy(edges_hbm.at[1, wid], idx_d)
    pltpu.sync_copy(ones_hbm, onesb)
    row0 = s * ZR
    pltpu.sync_copy(zeros_hbm, acc.at[pl.ds(row0, ZR)])
    plsc.subcore_barrier()

    def body(i, carry):
        pltpu.sync_copy(onesb, acc.at[idx_d.at[i]], add=True)
        return carry

    lax.fori_loop(0, CH, body, 0)
    plsc.subcore_barrier()
    w0 = s * 624

    @pl.when(c == 0)
    def _():
        pltpu.sync_copy(acc.at[pl.ds(w0, 624)], out0_hbm.at[pl.ds(w0, 624)])

        @pl.when(s == NS - 1)
        def _():
            pltpu.sync_copy(acc.at[pl.ds(9984, 16)],
                            out0_hbm.at[pl.ds(9984, 16)])

    @pl.when(c == 1)
    def _():
        pltpu.sync_copy(acc.at[pl.ds(w0, 624)], out1_hbm.at[pl.ds(w0, 624)])

        @pl.when(s == NS - 1)
        def _():
            pltpu.sync_copy(acc.at[pl.ds(9984, 16)],
                            out1_hbm.at[pl.ds(9984, 16)])


_sc_deg = functools.partial(
    pl.kernel,
    out_type=[jax.ShapeDtypeStruct((N, 16), jnp.float32),
              jax.ShapeDtypeStruct((N, 16), jnp.float32)],
    mesh=plsc.VectorSubcoreMesh(core_axis_name="c", subcore_axis_name="s"),
    scratch_types=[
        pltpu.VMEM((CH, K), jnp.int32),
        pltpu.VMEM((K, 16), jnp.float32),
        pltpu.VMEM_SHARED((NR, 16), jnp.float32),
        pltpu.SemaphoreType.DMA,
    ],
    compiler_params=pltpu.CompilerParams(use_tc_tiling_on_sc=False),
)(_sc_deg_body)


def _tc_b_body(deg0_ref, deg1_ref, x_ref, w1_ref, dis_ref, hs_ref):
    i = pl.program_id(0)
    row = i * BN + lax.broadcasted_iota(jnp.int32, (BN, 1), 0)
    padc = jnp.where(row < NPAD, 1.0, 0.0)
    deg = deg0_ref[...] + deg1_ref[...] + (1.0 - padc)
    dis = lax.rsqrt(deg)
    dis_ref[...] = dis
    xw = jnp.dot(x_ref[...], w1_ref[...], preferred_element_type=jnp.float32)
    hs_ref[...] = xw * dis[:, 0:1]


_tc_b = pl.pallas_call(
    _tc_b_body,
    grid=(NBLK,),
    in_specs=[
        pl.BlockSpec((BN, 16), lambda i: (i, 0)),
        pl.BlockSpec((BN, 16), lambda i: (i, 0)),
        pl.BlockSpec((BN, D), lambda i: (i, 0)),
        pl.BlockSpec((D, H), lambda i: (0, 0)),
    ],
    out_specs=[
        pl.BlockSpec((BN, 16), lambda i: (i, 0)),
        pl.BlockSpec((BN, H), lambda i: (i, 0)),
    ],
    out_shape=[
        jax.ShapeDtypeStruct((N, 16), jnp.float32),
        jax.ShapeDtypeStruct((N, H), jnp.float32),
    ],
)


def _tc_d_body(p0_ref, p1_ref, hs1_ref, dis_ref, b1_ref, w2_ref, hs2_ref):
    i = pl.program_id(0)
    row = i * BN + lax.broadcasted_iota(jnp.int32, (BN, 1), 0)
    self_w = jnp.where(row < NPAD, 0.0, 1.0)
    dis = dis_ref[...][:, 0:1]
    acc = p0_ref[...] + p1_ref[...] + self_w * hs1_ref[...]
    h1 = jnp.maximum(acc * dis + b1_ref[...], 0.0)
    hw = jnp.dot(h1, w2_ref[...], preferred_element_type=jnp.float32)
    hs2_ref[...] = hw * dis


_tc_d = pl.pallas_call(
    _tc_d_body,
    grid=(NBLK,),
    in_specs=[
        pl.BlockSpec((BN, H), lambda i: (i, 0)),
        pl.BlockSpec((BN, H), lambda i: (i, 0)),
        pl.BlockSpec((BN, H), lambda i: (i, 0)),
        pl.BlockSpec((BN, 16), lambda i: (i, 0)),
        pl.BlockSpec((1, H), lambda i: (0, 0)),
        pl.BlockSpec((H, H), lambda i: (0, 0)),
    ],
    out_specs=pl.BlockSpec((BN, H), lambda i: (i, 0)),
    out_shape=jax.ShapeDtypeStruct((N, H), jnp.float32),
)


def _tc_f_body(q0_ref, q1_ref, hs2_ref, dis_ref, b2_ref, batch_ref,
               wo_ref, bo_ref, out_ref, sum_ref, cnt_ref):
    g = pl.program_id(0)
    row = g * BN + lax.broadcasted_iota(jnp.int32, (BN, 1), 0)
    self_w = jnp.where(row < NPAD, 0.0, 1.0)
    dis = dis_ref[...][:, 0:1]
    h2 = ((q0_ref[...] + q1_ref[...] + self_w * hs2_ref[...])
          * dis + b2_ref[...])
    bblk = batch_ref[0]
    gids = lax.broadcasted_iota(jnp.int32, (G, BN), 0)
    oh = (gids == bblk).astype(jnp.float32)
    psum = jnp.dot(oh, h2, preferred_element_type=jnp.float32)
    pcnt = jnp.broadcast_to(jnp.sum(oh, axis=1, keepdims=True), (G, H))

    @pl.when(g == 0)
    def _():
        sum_ref[...] = jnp.zeros_like(sum_ref)
        cnt_ref[...] = jnp.zeros_like(cnt_ref)

    sum_ref[...] += psum
    cnt_ref[...] += pcnt

    @pl.when(g == NBLK - 1)
    def _():
        pooled = sum_ref[...] / jnp.maximum(cnt_ref[...], 1.0)
        out_ref[...] = (
            jnp.dot(pooled, wo_ref[...], preferred_element_type=jnp.float32)
            + bo_ref[...]
        )


_tc_f = pl.pallas_call(
    _tc_f_body,
    grid=(NBLK,),
    in_specs=[
        pl.BlockSpec((BN, H), lambda i: (i, 0)),
        pl.BlockSpec((BN, H), lambda i: (i, 0)),
        pl.BlockSpec((BN, H), lambda i: (i, 0)),
        pl.BlockSpec((BN, 16), lambda i: (i, 0)),
        pl.BlockSpec((1, H), lambda i: (0, 0)),
        pl.BlockSpec((1, 1, BN), lambda i: (i, 0, 0)),
        pl.BlockSpec((H, 128), lambda i: (0, 0)),
        pl.BlockSpec((1, 128), lambda i: (0, 0)),
    ],
    out_specs=pl.BlockSpec((G, 128), lambda i: (0, 0)),
    out_shape=jax.ShapeDtypeStruct((G, 128), jnp.float32),
    scratch_shapes=[
        pltpu.VMEM((G, H), jnp.float32),
        pltpu.VMEM((G, H), jnp.float32),
    ],
)


def kernel(x, edge_index, batch, W1, b1, W2, b2, Wo, bo):
    pad_idx = jnp.broadcast_to(jnp.arange(NPAD, dtype=jnp.int32), (2, NPAD))
    edges = jnp.concatenate([edge_index, pad_idx],
                            axis=1).reshape(2, NW, CH, K)
    zeros_h = jnp.zeros((ZR, H), jnp.float32)
    zeros16 = jnp.zeros((ZR, 16), jnp.float32)
    ones16 = jnp.ones((K, 16), jnp.float32)

    deg0, deg1 = _sc_deg(edges, ones16, zeros16)
    dis16, hs1 = _tc_b(deg0, deg1, x, W1)
    p0, p1 = _sc_scatter(edges, hs1, zeros_h)
    hs2 = _tc_d(p0, p1, hs1, dis16, b1.reshape(1, H), W2)
    q0, q1 = _sc_scatter(edges, hs2, zeros_h)
    wo_pad = jnp.pad(Wo, ((0, 0), (0, 128 - C_OUT)))
    bo_pad = jnp.pad(bo, (0, 128 - C_OUT)).reshape(1, 128)
    outp = _tc_f(q0, q1, hs2, dis16, b2.reshape(1, H),
                 batch.reshape(NBLK, 1, BN), wo_pad, bo_pad)
    return outp[:, :C_OUT]


# speedup vs baseline: 3.3815x; 1.0253x over previous
import functools

import jax
import jax.numpy as jnp
from jax import lax
from jax.experimental import pallas as pl
from jax.experimental.pallas import tpu as pltpu
from jax.experimental.pallas import tpu_sc as plsc

N = 10000
E = 320000
D = 128
H = 128
G = 64
C_OUT = 19

NC = 2
NS = 16
NW = NC * NS
K = 128
CH = 80
CHS = CH // 2
EPAD = NW * CH * K
NPAD = EPAD - E
NR = N
ZR = NR // NS
BN = 2000
NBLK = N // BN


def _sc_scatter_body(edges_hbm, hs_hbm, zeros_hbm, out0_hbm, out1_hbm,
                     idx_s, idx_d, rows_a, rows_b, acc, sem_a, sem_b, sem_c):
    c = lax.axis_index("c")
    s = lax.axis_index("s")
    wid = c * NS + s
    row0 = s * ZR
    pltpu.async_copy(edges_hbm.at[1, wid], idx_d, sem_a)
    pltpu.async_copy(zeros_hbm, acc.at[pl.ds(row0, ZR)], sem_c)
    pltpu.make_async_copy(edges_hbm.at[1, wid], idx_d, sem_a).wait()
    pltpu.make_async_copy(zeros_hbm, acc.at[pl.ds(row0, ZR)], sem_c).wait()
    plsc.subcore_barrier()

    for g in range(CH // CHS):
        base = g * CHS
        pltpu.sync_copy(edges_hbm.at[0, wid, pl.ds(base, CHS)], idx_s)
        pltpu.async_copy(hs_hbm.at[idx_s.at[0]], rows_a, sem_a)

        def body(j, carry):
            i0 = 2 * j
            pltpu.async_copy(hs_hbm.at[idx_s.at[i0 + 1]], rows_b, sem_b)
            pltpu.make_async_copy(hs_hbm.at[idx_s.at[i0]], rows_a,
                                  sem_a).wait()
            pltpu.sync_copy(rows_a, acc.at[idx_d.at[base + i0]], add=True)

            @pl.when(j < CHS // 2 - 1)
            def _():
                pltpu.async_copy(hs_hbm.at[idx_s.at[i0 + 2]], rows_a, sem_a)

            pltpu.make_async_copy(hs_hbm.at[idx_s.at[i0 + 1]], rows_b,
                                  sem_b).wait()
            pltpu.sync_copy(rows_b, acc.at[idx_d.at[base + i0 + 1]], add=True)
            return carry

        lax.fori_loop(0, CHS // 2, body, 0)

    plsc.subcore_barrier()
    w0 = s * 624

    @pl.when(c == 0)
    def _():
        pltpu.sync_copy(acc.at[pl.ds(w0, 624)], out0_hbm.at[pl.ds(w0, 624)])

        @pl.when(s == NS - 1)
        def _():
            pltpu.sync_copy(acc.at[pl.ds(9984, 16)],
                            out0_hbm.at[pl.ds(9984, 16)])

    @pl.when(c == 1)
    def _():
        pltpu.sync_copy(acc.at[pl.ds(w0, 624)], out1_hbm.at[pl.ds(w0, 624)])

        @pl.when(s == NS - 1)
        def _():
            pltpu.sync_copy(acc.at[pl.ds(9984, 16)],
                            out1_hbm.at[pl.ds(9984, 16)])


_sc_scatter = functools.partial(
    pl.kernel,
    out_type=[jax.ShapeDtypeStruct((N, H), jnp.float32),
              jax.ShapeDtypeStruct((N, H), jnp.float32)],
    mesh=plsc.VectorSubcoreMesh(core_axis_name="c", subcore_axis_name="s"),
    scratch_types=[
        pltpu.VMEM((CHS, K), jnp.int32),
        pltpu.VMEM((CH, K), jnp.int32),
        pltpu.VMEM((K, H), jnp.float32),
        pltpu.VMEM((K, H), jnp.float32),
        pltpu.VMEM_SHARED((NR, H), jnp.float32),
        pltpu.SemaphoreType.DMA,
        pltpu.SemaphoreType.DMA,
        pltpu.SemaphoreType.DMA,
    ],
    compiler_params=pltpu.CompilerParams(use_tc_tiling_on_sc=False),
)(_sc_scatter_body)


def _sc_deg_body(edges_hbm, ones_hbm, zeros_hbm, out0_hbm, out1_hbm,
                 idx_d, onesb, acc, sem):
    c = lax.axis_index("c")
    s = lax.axis_index("s")
    wid = c * NS + s
    pltpu.sync_copy(edges_hbm.at[1, wid], idx_d)
    pltpu.sync_copy(ones_hbm, onesb)
    row0 = s * ZR
    pltpu.sync_copy(zeros_hbm, acc.at[pl.ds(row0, ZR)])
    plsc.subcore_barrier()

    def body(i, carry):
        pltpu.sync_copy(onesb, acc.at[idx_d.at[i]], add=True)
        return carry

    lax.fori_loop(0, CH, body, 0)
    plsc.subcore_barrier()
    w0 = s * 624

    @pl.when(c == 0)
    def _():
        pltpu.sync_copy(acc.at[pl.ds(w0, 624)], out0_hbm.at[pl.ds(w0, 624)])

        @pl.when(s == NS - 1)
        def _():
            pltpu.sync_copy(acc.at[pl.ds(9984, 16)],
                            out0_hbm.at[pl.ds(9984, 16)])

    @pl.when(c == 1)
    def _():
        pltpu.sync_copy(acc.at[pl.ds(w0, 624)], out1_hbm.at[pl.ds(w0, 624)])

        @pl.when(s == NS - 1)
        def _():
            pltpu.sync_copy(acc.at[pl.ds(9984, 16)],
                            out1_hbm.at[pl.ds(9984, 16)])


_sc_deg = functools.partial(
    pl.kernel,
    out_type=[jax.ShapeDtypeStruct((N, 16), jnp.float32),
              jax.ShapeDtypeStruct((N, 16), jnp.float32)],
    mesh=plsc.VectorSubcoreMesh(core_axis_name="c", subcore_axis_name="s"),
    scratch_types=[
        pltpu.VMEM((CH, K), jnp.int32),
        pltpu.VMEM((K, 16), jnp.float32),
        pltpu.VMEM_SHARED((NR, 16), jnp.float32),
        pltpu.SemaphoreType.DMA,
    ],
    compiler_params=pltpu.CompilerParams(use_tc_tiling_on_sc=False),
)(_sc_deg_body)


def _tc_b_body(deg0_ref, deg1_ref, x_ref, w1_ref, dis_ref, hs_ref):
    i = pl.program_id(0)
    row = i * BN + lax.broadcasted_iota(jnp.int32, (BN, 1), 0)
    padc = jnp.where(row < NPAD, 1.0, 0.0)
    deg = deg0_ref[...] + deg1_ref[...] + (1.0 - padc)
    dis = lax.rsqrt(deg)
    dis_ref[...] = dis
    xw = jnp.dot(x_ref[...], w1_ref[...], preferred_element_type=jnp.float32)
    hs_ref[...] = xw * dis[:, 0:1]


_tc_b = pl.pallas_call(
    _tc_b_body,
    grid=(NBLK,),
    in_specs=[
        pl.BlockSpec((BN, 16), lambda i: (i, 0)),
        pl.BlockSpec((BN, 16), lambda i: (i, 0)),
        pl.BlockSpec((BN, D), lambda i: (i, 0)),
        pl.BlockSpec((D, H), lambda i: (0, 0)),
    ],
    out_specs=[
        pl.BlockSpec((BN, 16), lambda i: (i, 0)),
        pl.BlockSpec((BN, H), lambda i: (i, 0)),
    ],
    out_shape=[
        jax.ShapeDtypeStruct((N, 16), jnp.float32),
        jax.ShapeDtypeStruct((N, H), jnp.float32),
    ],
)


def _tc_d_body(p0_ref, p1_ref, hs1_ref, dis_ref, b1_ref, w2_ref, hs2_ref):
    i = pl.program_id(0)
    row = i * BN + lax.broadcasted_iota(jnp.int32, (BN, 1), 0)
    self_w = jnp.where(row < NPAD, 0.0, 1.0)
    dis = dis_ref[...][:, 0:1]
    acc = p0_ref[...] + p1_ref[...] + self_w * hs1_ref[...]
    h1 = jnp.maximum(acc * dis + b1_ref[...], 0.0)
    hw = jnp.dot(h1, w2_ref[...], preferred_element_type=jnp.float32)
    hs2_ref[...] = hw * dis


_tc_d = pl.pallas_call(
    _tc_d_body,
    grid=(NBLK,),
    in_specs=[
        pl.BlockSpec((BN, H), lambda i: (i, 0)),
        pl.BlockSpec((BN, H), lambda i: (i, 0)),
        pl.BlockSpec((BN, H), lambda i: (i, 0)),
        pl.BlockSpec((BN, 16), lambda i: (i, 0)),
        pl.BlockSpec((1, H), lambda i: (0, 0)),
        pl.BlockSpec((H, H), lambda i: (0, 0)),
    ],
    out_specs=pl.BlockSpec((BN, H), lambda i: (i, 0)),
    out_shape=jax.ShapeDtypeStruct((N, H), jnp.float32),
)


def _tc_f_body(q0_ref, q1_ref, hs2_ref, dis_ref, b2_ref, batch_ref,
               wo_ref, bo_ref, out_ref, sum_ref, cnt_ref):
    g = pl.program_id(0)
    row = g * BN + lax.broadcasted_iota(jnp.int32, (BN, 1), 0)
    self_w = jnp.where(row < NPAD, 0.0, 1.0)
    dis = dis_ref[...][:, 0:1]
    h2 = ((q0_ref[...] + q1_ref[...] + self_w * hs2_ref[...])
          * dis + b2_ref[...])
    bblk = batch_ref[0]
    gids = lax.broadcasted_iota(jnp.int32, (G, BN), 0)
    oh = (gids == bblk).astype(jnp.float32)
    psum = jnp.dot(oh, h2, preferred_element_type=jnp.float32)
    pcnt = jnp.broadcast_to(jnp.sum(oh, axis=1, keepdims=True), (G, H))

    @pl.when(g == 0)
    def _():
        sum_ref[...] = jnp.zeros_like(sum_ref)
        cnt_ref[...] = jnp.zeros_like(cnt_ref)

    sum_ref[...] += psum
    cnt_ref[...] += pcnt

    @pl.when(g == NBLK - 1)
    def _():
        pooled = sum_ref[...] / jnp.maximum(cnt_ref[...], 1.0)
        out_ref[...] = (
            jnp.dot(pooled, wo_ref[...], preferred_element_type=jnp.float32)
            + bo_ref[...]
        )


_tc_f = pl.pallas_call(
    _tc_f_body,
    grid=(NBLK,),
    in_specs=[
        pl.BlockSpec((BN, H), lambda i: (i, 0)),
        pl.BlockSpec((BN, H), lambda i: (i, 0)),
        pl.BlockSpec((BN, H), lambda i: (i, 0)),
        pl.BlockSpec((BN, 16), lambda i: (i, 0)),
        pl.BlockSpec((1, H), lambda i: (0, 0)),
        pl.BlockSpec((1, 1, BN), lambda i: (i, 0, 0)),
        pl.BlockSpec((H, 128), lambda i: (0, 0)),
        pl.BlockSpec((1, 128), lambda i: (0, 0)),
    ],
    out_specs=pl.BlockSpec((G, 128), lambda i: (0, 0)),
    out_shape=jax.ShapeDtypeStruct((G, 128), jnp.float32),
    scratch_shapes=[
        pltpu.VMEM((G, H), jnp.float32),
        pltpu.VMEM((G, H), jnp.float32),
    ],
)


def kernel(x, edge_index, batch, W1, b1, W2, b2, Wo, bo):
    pad_idx = jnp.broadcast_to(jnp.arange(NPAD, dtype=jnp.int32), (2, NPAD))
    edges = jnp.concatenate([edge_index, pad_idx],
                            axis=1).reshape(2, NW, CH, K)
    zeros_h = jnp.zeros((ZR, H), jnp.float32)
    zeros16 = jnp.zeros((ZR, 16), jnp.float32)
    ones16 = jnp.ones((K, 16), jnp.float32)

    deg0, deg1 = _sc_deg(edges, ones16, zeros16)
    dis16, hs1 = _tc_b(deg0, deg1, x, W1)
    p0, p1 = _sc_scatter(edges, hs1, zeros_h)
    hs2 = _tc_d(p0, p1, hs1, dis16, b1.reshape(1, H), W2)
    q0, q1 = _sc_scatter(edges, hs2, zeros_h)
    wo_pad = jnp.pad(Wo, ((0, 0), (0, 128 - C_OUT)))
    bo_pad = jnp.pad(bo, (0, 128 - C_OUT)).reshape(1, 128)
    outp = _tc_f(q0, q1, hs2, dis16, b2.reshape(1, H),
                 batch.reshape(NBLK, 1, BN), wo_pad, bo_pad)
    return outp[:, :C_OUT]


# speedup vs baseline: 3.4184x; 1.0109x over previous
import functools

import jax
import jax.numpy as jnp
from jax import lax
from jax.experimental import pallas as pl
from jax.experimental.pallas import tpu as pltpu
from jax.experimental.pallas import tpu_sc as plsc

N = 10000
E = 320000
D = 128
H = 128
G = 64
C_OUT = 19

NC = 2
NS = 16
NW = NC * NS
K = 128
CH = 80
CHS = CH // 2
EPAD = NW * CH * K
NPAD = EPAD - E
NR = N
ZR = NR // NS
BN = 5000
NBLK = N // BN


def _sc_scatter_body(edges_hbm, hs_hbm, zeros_hbm, out0_hbm, out1_hbm,
                     idx_s, idx_d, rows_a, rows_b, acc, sem_a, sem_b, sem_c):
    c = lax.axis_index("c")
    s = lax.axis_index("s")
    wid = c * NS + s
    row0 = s * ZR
    pltpu.async_copy(edges_hbm.at[1, wid], idx_d, sem_a)
    pltpu.async_copy(zeros_hbm, acc.at[pl.ds(row0, ZR)], sem_c)
    pltpu.make_async_copy(edges_hbm.at[1, wid], idx_d, sem_a).wait()
    pltpu.make_async_copy(zeros_hbm, acc.at[pl.ds(row0, ZR)], sem_c).wait()
    plsc.subcore_barrier()

    for g in range(CH // CHS):
        base = g * CHS
        pltpu.sync_copy(edges_hbm.at[0, wid, pl.ds(base, CHS)], idx_s)
        pltpu.async_copy(hs_hbm.at[idx_s.at[0]], rows_a, sem_a)

        def body(j, carry):
            i0 = 2 * j
            pltpu.async_copy(hs_hbm.at[idx_s.at[i0 + 1]], rows_b, sem_b)
            pltpu.make_async_copy(hs_hbm.at[idx_s.at[i0]], rows_a,
                                  sem_a).wait()
            pltpu.sync_copy(rows_a, acc.at[idx_d.at[base + i0]], add=True)

            @pl.when(j < CHS // 2 - 1)
            def _():
                pltpu.async_copy(hs_hbm.at[idx_s.at[i0 + 2]], rows_a, sem_a)

            pltpu.make_async_copy(hs_hbm.at[idx_s.at[i0 + 1]], rows_b,
                                  sem_b).wait()
            pltpu.sync_copy(rows_b, acc.at[idx_d.at[base + i0 + 1]], add=True)
            return carry

        lax.fori_loop(0, CHS // 2, body, 0)

    plsc.subcore_barrier()
    w0 = s * 624

    @pl.when(c == 0)
    def _():
        pltpu.sync_copy(acc.at[pl.ds(w0, 624)], out0_hbm.at[pl.ds(w0, 624)])

        @pl.when(s == NS - 1)
        def _():
            pltpu.sync_copy(acc.at[pl.ds(9984, 16)],
                            out0_hbm.at[pl.ds(9984, 16)])

    @pl.when(c == 1)
    def _():
        pltpu.sync_copy(acc.at[pl.ds(w0, 624)], out1_hbm.at[pl.ds(w0, 624)])

        @pl.when(s == NS - 1)
        def _():
            pltpu.sync_copy(acc.at[pl.ds(9984, 16)],
                            out1_hbm.at[pl.ds(9984, 16)])


_sc_scatter = functools.partial(
    pl.kernel,
    out_type=[jax.ShapeDtypeStruct((N, H), jnp.float32),
              jax.ShapeDtypeStruct((N, H), jnp.float32)],
    mesh=plsc.VectorSubcoreMesh(core_axis_name="c", subcore_axis_name="s"),
    scratch_types=[
        pltpu.VMEM((CHS, K), jnp.int32),
        pltpu.VMEM((CH, K), jnp.int32),
        pltpu.VMEM((K, H), jnp.float32),
        pltpu.VMEM((K, H), jnp.float32),
        pltpu.VMEM_SHARED((NR, H), jnp.float32),
        pltpu.SemaphoreType.DMA,
        pltpu.SemaphoreType.DMA,
        pltpu.SemaphoreType.DMA,
    ],
    compiler_params=pltpu.CompilerParams(use_tc_tiling_on_sc=False),
)(_sc_scatter_body)


def _sc_deg_body(edges_hbm, ones_hbm, zeros_hbm, out0_hbm, out1_hbm,
                 idx_d, onesb, acc, sem):
    c = lax.axis_index("c")
    s = lax.axis_index("s")
    wid = c * NS + s
    pltpu.sync_copy(edges_hbm.at[1, wid], idx_d)
    pltpu.sync_copy(ones_hbm, onesb)
    row0 = s * ZR
    pltpu.sync_copy(zeros_hbm, acc.at[pl.ds(row0, ZR)])
    plsc.subcore_barrier()

    def body(i, carry):
        pltpu.sync_copy(onesb, acc.at[idx_d.at[i]], add=True)
        return carry

    lax.fori_loop(0, CH, body, 0)
    plsc.subcore_barrier()
    w0 = s * 624

    @pl.when(c == 0)
    def _():
        pltpu.sync_copy(acc.at[pl.ds(w0, 624)], out0_hbm.at[pl.ds(w0, 624)])

        @pl.when(s == NS - 1)
        def _():
            pltpu.sync_copy(acc.at[pl.ds(9984, 16)],
                            out0_hbm.at[pl.ds(9984, 16)])

    @pl.when(c == 1)
    def _():
        pltpu.sync_copy(acc.at[pl.ds(w0, 624)], out1_hbm.at[pl.ds(w0, 624)])

        @pl.when(s == NS - 1)
        def _():
            pltpu.sync_copy(acc.at[pl.ds(9984, 16)],
                            out1_hbm.at[pl.ds(9984, 16)])


_sc_deg = functools.partial(
    pl.kernel,
    out_type=[jax.ShapeDtypeStruct((N, 16), jnp.float32),
              jax.ShapeDtypeStruct((N, 16), jnp.float32)],
    mesh=plsc.VectorSubcoreMesh(core_axis_name="c", subcore_axis_name="s"),
    scratch_types=[
        pltpu.VMEM((CH, K), jnp.int32),
        pltpu.VMEM((K, 16), jnp.float32),
        pltpu.VMEM_SHARED((NR, 16), jnp.float32),
        pltpu.SemaphoreType.DMA,
    ],
    compiler_params=pltpu.CompilerParams(use_tc_tiling_on_sc=False),
)(_sc_deg_body)


def _tc_b_body(deg0_ref, deg1_ref, x_ref, w1_ref, dis_ref, hs_ref):
    i = pl.program_id(0)
    row = i * BN + lax.broadcasted_iota(jnp.int32, (BN, 1), 0)
    padc = jnp.where(row < NPAD, 1.0, 0.0)
    deg = deg0_ref[...] + deg1_ref[...] + (1.0 - padc)
    dis = lax.rsqrt(deg)
    dis_ref[...] = dis
    xw = jnp.dot(x_ref[...], w1_ref[...], preferred_element_type=jnp.float32)
    hs_ref[...] = xw * dis[:, 0:1]


_tc_b = pl.pallas_call(
    _tc_b_body,
    grid=(NBLK,),
    in_specs=[
        pl.BlockSpec((BN, 16), lambda i: (i, 0)),
        pl.BlockSpec((BN, 16), lambda i: (i, 0)),
        pl.BlockSpec((BN, D), lambda i: (i, 0)),
        pl.BlockSpec((D, H), lambda i: (0, 0)),
    ],
    out_specs=[
        pl.BlockSpec((BN, 16), lambda i: (i, 0)),
        pl.BlockSpec((BN, H), lambda i: (i, 0)),
    ],
    out_shape=[
        jax.ShapeDtypeStruct((N, 16), jnp.float32),
        jax.ShapeDtypeStruct((N, H), jnp.float32),
    ],
)


def _tc_d_body(p0_ref, p1_ref, hs1_ref, dis_ref, b1_ref, w2_ref, hs2_ref):
    i = pl.program_id(0)
    row = i * BN + lax.broadcasted_iota(jnp.int32, (BN, 1), 0)
    self_w = jnp.where(row < NPAD, 0.0, 1.0)
    dis = dis_ref[...][:, 0:1]
    acc = p0_ref[...] + p1_ref[...] + self_w * hs1_ref[...]
    h1 = jnp.maximum(acc * dis + b1_ref[...], 0.0)
    hw = jnp.dot(h1, w2_ref[...], preferred_element_type=jnp.float32)
    hs2_ref[...] = hw * dis


_tc_d = pl.pallas_call(
    _tc_d_body,
    grid=(NBLK,),
    in_specs=[
        pl.BlockSpec((BN, H), lambda i: (i, 0)),
        pl.BlockSpec((BN, H), lambda i: (i, 0)),
        pl.BlockSpec((BN, H), lambda i: (i, 0)),
        pl.BlockSpec((BN, 16), lambda i: (i, 0)),
        pl.BlockSpec((1, H), lambda i: (0, 0)),
        pl.BlockSpec((H, H), lambda i: (0, 0)),
    ],
    out_specs=pl.BlockSpec((BN, H), lambda i: (i, 0)),
    out_shape=jax.ShapeDtypeStruct((N, H), jnp.float32),
)


def _tc_f_body(q0_ref, q1_ref, hs2_ref, dis_ref, b2_ref, batch_ref,
               wo_ref, bo_ref, out_ref, sum_ref, cnt_ref):
    g = pl.program_id(0)
    row = g * BN + lax.broadcasted_iota(jnp.int32, (BN, 1), 0)
    self_w = jnp.where(row < NPAD, 0.0, 1.0)
    dis = dis_ref[...][:, 0:1]
    h2 = ((q0_ref[...] + q1_ref[...] + self_w * hs2_ref[...])
          * dis + b2_ref[...])
    bblk = batch_ref[0]
    gids = lax.broadcasted_iota(jnp.int32, (G, BN), 0)
    oh = (gids == bblk).astype(jnp.float32)
    psum = jnp.dot(oh, h2, preferred_element_type=jnp.float32)
    pcnt = jnp.broadcast_to(jnp.sum(oh, axis=1, keepdims=True), (G, H))

    @pl.when(g == 0)
    def _():
        sum_ref[...] = jnp.zeros_like(sum_ref)
        cnt_ref[...] = jnp.zeros_like(cnt_ref)

    sum_ref[...] += psum
    cnt_ref[...] += pcnt

    @pl.when(g == NBLK - 1)
    def _():
        pooled = sum_ref[...] / jnp.maximum(cnt_ref[...], 1.0)
        out_ref[...] = (
            jnp.dot(pooled, wo_ref[...], preferred_element_type=jnp.float32)
            + bo_ref[...]
        )


_tc_f = pl.pallas_call(
    _tc_f_body,
    grid=(NBLK,),
    in_specs=[
        pl.BlockSpec((BN, H), lambda i: (i, 0)),
        pl.BlockSpec((BN, H), lambda i: (i, 0)),
        pl.BlockSpec((BN, H), lambda i: (i, 0)),
        pl.BlockSpec((BN, 16), lambda i: (i, 0)),
        pl.BlockSpec((1, H), lambda i: (0, 0)),
        pl.BlockSpec((1, 1, BN), lambda i: (i, 0, 0)),
        pl.BlockSpec((H, 128), lambda i: (0, 0)),
        pl.BlockSpec((1, 128), lambda i: (0, 0)),
    ],
    out_specs=pl.BlockSpec((G, 128), lambda i: (0, 0)),
    out_shape=jax.ShapeDtypeStruct((G, 128), jnp.float32),
    scratch_shapes=[
        pltpu.VMEM((G, H), jnp.float32),
        pltpu.VMEM((G, H), jnp.float32),
    ],
)


def kernel(x, edge_index, batch, W1, b1, W2, b2, Wo, bo):
    pad_idx = jnp.broadcast_to(jnp.arange(NPAD, dtype=jnp.int32), (2, NPAD))
    edges = jnp.concatenate([edge_index, pad_idx],
                            axis=1).reshape(2, NW, CH, K)
    zeros_h = jnp.zeros((ZR, H), jnp.float32)
    zeros16 = jnp.zeros((ZR, 16), jnp.float32)
    ones16 = jnp.ones((K, 16), jnp.float32)

    deg0, deg1 = _sc_deg(edges, ones16, zeros16)
    dis16, hs1 = _tc_b(deg0, deg1, x, W1)
    p0, p1 = _sc_scatter(edges, hs1, zeros_h)
    hs2 = _tc_d(p0, p1, hs1, dis16, b1.reshape(1, H), W2)
    q0, q1 = _sc_scatter(edges, hs2, zeros_h)
    wo_pad = jnp.pad(Wo, ((0, 0), (0, 128 - C_OUT)))
    bo_pad = jnp.pad(bo, (0, 128 - C_OUT)).reshape(1, 128)
    outp = _tc_f(q0, q1, hs2, dis16, b2.reshape(1, H),
                 batch.reshape(NBLK, 1, BN), wo_pad, bo_pad)
    return outp[:, :C_OUT]
